# baseline (speedup 1.0000x reference)
import jax
import jax.numpy as jnp
from jax import lax
from jax.experimental import pallas as pl
from jax.experimental.pallas import tpu as pltpu
from jax.experimental.pallas import tpu_sc as plsc

B, C, T, D = 8, 64, 1024, 64
QL, NCLS, NCH, NSUB = 256, 1000, 64, 1000
NC, NS = 2, 16
CHUNK = 128
NTJ = T // CHUNK


def _body(x_hbm, ids_hbm, sid_hbm, cond_hbm, quant_hbm, condz_hbm, ch_hbm,
          sub_hbm, out_hbm,
          idsv, sidv, che_all, sub_all, cidx, cese, xbuf, buf1, gsem, wsem):
    w = lax.axis_index("s") * NC + lax.axis_index("c")
    b = w // 4
    c0 = (w % 4) * 16

    pltpu.sync_copy(ids_hbm, idsv)
    pltpu.sync_copy(sid_hbm, sidv)
    pltpu.sync_copy(ch_hbm.at[idsv], che_all)
    pltpu.sync_copy(sub_hbm.at[sidv], sub_all)
    pltpu.sync_copy(cond_hbm.at[b], cidx)
    for j in range(NTJ):
        pltpu.async_copy(condz_hbm.at[cidx.at[j]],
                         cese.at[pl.ds(j * CHUNK, CHUNK)], gsem.at[j & 1])
    for j in range(NTJ):
        pltpu.make_async_copy(condz_hbm.at[cidx.at[j]],
                              cese.at[pl.ds(j * CHUNK, CHUNK)],
                              gsem.at[j & 1]).wait()
    se_v = [sub_all[b, pl.ds(16 * k, 16)] for k in range(4)]

    def gather(i, p, t0):
        return pltpu.make_async_copy(
            quant_hbm.at[xbuf.at[i]], buf1.at[p], gsem.at[p])

    def write(i, p, t0):
        return pltpu.make_async_copy(
            buf1.at[p], out_hbm.at[b * C + c0 + i, pl.ds(t0, CHUNK)],
            wsem.at[p])

    def tj_body(j, carry):
        t0 = j * CHUNK
        pltpu.sync_copy(x_hbm.at[b, pl.ds(c0, 16), pl.ds(t0, CHUNK)], xbuf)
        gather(0, 0, t0).start()

        def c_body(i, carry2):
            p = i & 1
            c = c0 + i
            addc = [che_all[c, pl.ds(16 * k, 16)] + se_v[k] for k in range(4)]

            @pl.when(i < 15)
            def _prefetch():
                @pl.when(i >= 1)
                def _():
                    write(i - 1, 1 - p, t0).wait()
                gather(i + 1, 1 - p, t0).start()

            gather(i, p, t0).wait()

            def add_body(rr, carry3):
                for u in range(2):
                    t = t0 + 2 * rr + u
                    for k in range(4):
                        plsc.addupdate(
                            buf1.at[p, 2 * rr + u, pl.ds(16 * k, 16)],
                            cese[t, pl.ds(16 * k, 16)] + addc[k])
                return carry3

            lax.fori_loop(0, CHUNK // 2, add_body, 0)
            write(i, p, t0).start()
            return carry2

        lax.fori_loop(0, 16, c_body, 0)
        write(14, 0, t0).wait()
        write(15, 1, t0).wait()
        return carry

    lax.fori_loop(0, NTJ, tj_body, 0)


def kernel(x, ids, cond, sid, quant_table, cond_table, ch_table, sub_table):
    x32 = x.astype(jnp.int32)
    ids32 = ids.astype(jnp.int32)
    cond32 = cond.reshape(B, NTJ, CHUNK).astype(jnp.int32)
    sid32 = jnp.pad(sid.reshape(B).astype(jnp.int32), (0, 8))
    condz = cond_table.at[0].set(0.0)

    p = pl.kernel(
        _body,
        out_type=jax.ShapeDtypeStruct((B * C, T, D), jnp.float32),
        mesh=plsc.VectorSubcoreMesh(core_axis_name="c", subcore_axis_name="s",
                                    num_cores=NC, num_subcores=NS),
        compiler_params=pltpu.CompilerParams(use_tc_tiling_on_sc=False),
        scratch_types=[
            pltpu.VMEM((NCH,), jnp.int32),
            pltpu.VMEM((16,), jnp.int32),
            pltpu.VMEM((NCH, D), jnp.float32),
            pltpu.VMEM((16, D), jnp.float32),
            pltpu.VMEM((NTJ, CHUNK), jnp.int32),
            pltpu.VMEM((T, D), jnp.float32),
            pltpu.VMEM((16, CHUNK), jnp.int32),
            pltpu.VMEM((2, CHUNK, D), jnp.float32),
            pltpu.SemaphoreType.DMA((2,)),
            pltpu.SemaphoreType.DMA((2,)),
        ],
    )
    return p(x32, ids32, sid32, cond32, quant_table, condz, ch_table,
             sub_table)


# speedup vs baseline: 1.1082x; 1.1082x over previous
import jax
import jax.numpy as jnp
from jax import lax
from jax.experimental import pallas as pl
from jax.experimental.pallas import tpu as pltpu
from jax.experimental.pallas import tpu_sc as plsc

B, C, T, D = 8, 64, 1024, 64
QL, NCLS, NCH, NSUB = 256, 1000, 64, 1000
NC, NS = 2, 16
CHUNK = 128
NTJ = T // CHUNK


def _body(x_hbm, ids_hbm, sid_hbm, cond_hbm, quant_hbm, condz_hbm, ch_hbm,
          sub_hbm, out_hbm,
          idsv, sidv, che_all, sub_all, cidx, cese, xbuf, buf1, gsem, wsem):
    w = lax.axis_index("s") * NC + lax.axis_index("c")
    b = w // 4
    c0 = (w % 4) * 16

    pltpu.sync_copy(ids_hbm, idsv)
    pltpu.sync_copy(sid_hbm, sidv)
    pltpu.sync_copy(ch_hbm.at[idsv], che_all)
    pltpu.sync_copy(sub_hbm.at[sidv], sub_all)
    pltpu.sync_copy(cond_hbm.at[b], cidx)
    for j in range(NTJ):
        pltpu.async_copy(condz_hbm.at[cidx.at[j]],
                         cese.at[pl.ds(j * CHUNK, CHUNK)], gsem.at[j & 1])
    for j in range(NTJ):
        pltpu.make_async_copy(condz_hbm.at[cidx.at[j]],
                              cese.at[pl.ds(j * CHUNK, CHUNK)],
                              gsem.at[j & 1]).wait()
    se_v = [sub_all[b, pl.ds(16 * k, 16)] for k in range(4)]

    def gather(i, p, t0):
        return pltpu.make_async_copy(
            quant_hbm.at[xbuf.at[i]], buf1.at[p], gsem.at[p])

    def write(i, p, t0):
        return pltpu.make_async_copy(
            buf1.at[p], out_hbm.at[b * C + c0 + i, pl.ds(t0, CHUNK)],
            wsem.at[p])

    def tj_body(j, carry):
        t0 = j * CHUNK
        pltpu.sync_copy(x_hbm.at[b, pl.ds(c0, 16), pl.ds(t0, CHUNK)], xbuf)
        gather(0, 0, t0).start()

        def c_body(i, carry2):
            p = i & 1
            c = c0 + i
            addc = [che_all[c, pl.ds(16 * k, 16)] + se_v[k] for k in range(4)]

            @pl.when(i < 15)
            def _prefetch():
                @pl.when(i >= 1)
                def _():
                    write(i - 1, 1 - p, t0).wait()
                gather(i + 1, 1 - p, t0).start()

            gather(i, p, t0).wait()

            @plsc.parallel_loop(0, CHUNK, 1, unroll=8)
            def add_body(rr):
                t = t0 + rr
                for k in range(4):
                    plsc.addupdate(buf1.at[p, rr, pl.ds(16 * k, 16)],
                                   cese[t, pl.ds(16 * k, 16)] + addc[k])
            write(i, p, t0).start()
            return carry2

        lax.fori_loop(0, 16, c_body, 0)
        write(14, 0, t0).wait()
        write(15, 1, t0).wait()
        return carry

    lax.fori_loop(0, NTJ, tj_body, 0)


def kernel(x, ids, cond, sid, quant_table, cond_table, ch_table, sub_table):
    x32 = x.astype(jnp.int32)
    ids32 = ids.astype(jnp.int32)
    cond32 = cond.reshape(B, NTJ, CHUNK).astype(jnp.int32)
    sid32 = jnp.pad(sid.reshape(B).astype(jnp.int32), (0, 8))
    condz = cond_table.at[0].set(0.0)

    p = pl.kernel(
        _body,
        out_type=jax.ShapeDtypeStruct((B * C, T, D), jnp.float32),
        mesh=plsc.VectorSubcoreMesh(core_axis_name="c", subcore_axis_name="s",
                                    num_cores=NC, num_subcores=NS),
        compiler_params=pltpu.CompilerParams(use_tc_tiling_on_sc=False),
        scratch_types=[
            pltpu.VMEM((NCH,), jnp.int32),
            pltpu.VMEM((16,), jnp.int32),
            pltpu.VMEM((NCH, D), jnp.float32),
            pltpu.VMEM((16, D), jnp.float32),
            pltpu.VMEM((NTJ, CHUNK), jnp.int32),
            pltpu.VMEM((T, D), jnp.float32),
            pltpu.VMEM((16, CHUNK), jnp.int32),
            pltpu.VMEM((2, CHUNK, D), jnp.float32),
            pltpu.SemaphoreType.DMA((2,)),
            pltpu.SemaphoreType.DMA((2,)),
        ],
    )
    return p(x32, ids32, sid32, cond32, quant_table, condz, ch_table,
             sub_table)
